# Initial kernel scaffold; baseline (speedup 1.0000x reference)
import jax
import jax.numpy as jnp
from jax.experimental import pallas as pl


def kernel(x, edge_index, edge_types, ggnn_W, ggnn_b, gru_w_ih, gru_b_ih, gru_w_hh, gru_b_hh, conv1_w, conv1_b, conv2_w, conv2_b, cconv1_w, cconv1_b, cconv2_w, cconv2_b, mlpy_w, mlpy_b, mlpz_w, mlpz_b, l1_w, l1_b, f1_w, f1_b, f2_w, f2_b, cls_w, cls_b):
    raise NotImplementedError("write your pallas kernel here")


# speedup vs baseline: 3.5241x; 3.5241x over previous
import functools

import jax
import jax.numpy as jnp
from jax import lax
from jax.experimental import pallas as pl
from jax.experimental.pallas import tpu as pltpu
from jax.experimental.pallas import tpu_sc as plsc

N = 10000
E = 320000
IN_F = 128
H = 200
HH = H // 2
PH = 112
STEPS = 8
CONCAT = IN_F + H

NC = 2
NS = 16
NW = NC * NS
EPT = E // NS
K = 80
NI = EPT // K
ROW_BLK = 632
ACC_ROWS = NS * ROW_BLK
LAST_ROWS = N - (NS - 1) * ROW_BLK

_F32 = jnp.float32


def _mm(a, b):
    return lax.dot_general(a, b, (((1,), (0,)), ((), ())),
                           preferred_element_type=_F32)


def _sc_scatter_body(t_hbm, gidx_hbm, dst_hbm, out_hbm,
                     acc, mg0, md0, st0, sem0):
    c = lax.axis_index("c")
    s = lax.axis_index("s")
    wid = c * NS + s
    row0 = pl.multiple_of(s * ROW_BLK, 8)

    def zr(r, carry):
        for j in range(PH // 16):
            st0[r, pl.ds(j * 16, 16)] = jnp.zeros((16,), _F32)
        return carry
    lax.fori_loop(0, K, zr, 0)
    for t in range(ROW_BLK // K):
        pltpu.sync_copy(st0, acc.at[pl.ds(row0 + t * K, K)])
    rem = ROW_BLK % K
    if rem:
        pltpu.sync_copy(st0.at[pl.ds(0, rem)],
                        acc.at[pl.ds(row0 + (ROW_BLK // K) * K, rem)])
    plsc.subcore_barrier()

    def step(i, carry):
        pltpu.sync_copy(gidx_hbm.at[wid, i], mg0)
        pltpu.sync_copy(dst_hbm.at[wid, i], md0)
        pltpu.async_copy(t_hbm.at[mg0], st0, sem0).wait()
        pltpu.sync_copy(st0, acc.at[md0], add=True)
        return carry
    lax.fori_loop(0, NI, step, 0)
    plsc.subcore_barrier()

    @pl.when(s < NS - 1)
    def _():
        pltpu.sync_copy(acc.at[pl.ds(row0, ROW_BLK)],
                        out_hbm.at[c, pl.ds(row0, ROW_BLK)])
    @pl.when(s == NS - 1)
    def _():
        pltpu.sync_copy(acc.at[pl.ds((NS - 1) * ROW_BLK, LAST_ROWS)],
                        out_hbm.at[c, pl.ds((NS - 1) * ROW_BLK, LAST_ROWS)])


@functools.cache
def _get_sc_scatter():
    return pl.kernel(
        _sc_scatter_body,
        out_type=jax.ShapeDtypeStruct((NC, N, PH), _F32),
        mesh=plsc.VectorSubcoreMesh(core_axis_name="c", subcore_axis_name="s",
                                    num_cores=NC, num_subcores=NS),
        scratch_types=[
            pltpu.VMEM_SHARED((ACC_ROWS, PH), _F32),
            pltpu.VMEM((K,), jnp.int32),
            pltpu.VMEM((K,), jnp.int32),
            pltpu.VMEM((K, PH), _F32),
            pltpu.SemaphoreType.DMA,
        ],
        compiler_params=pltpu.CompilerParams(use_tc_tiling_on_sc=False),
    )


_BN = 2000


def _emit_T(hnew, w_ref, b_ref, tout_ref):
    for q in range(4):
        tout_ref[q] = _mm(hnew, w_ref[q]) + b_ref[q:q + 1]


def _tinit_body(h_ref, w_ref, b_ref, tout_ref):
    _emit_T(h_ref[...], w_ref, b_ref, tout_ref)


def _tinit(h, w4, b4):
    return pl.pallas_call(
        _tinit_body,
        grid=(N // _BN,),
        in_specs=[
            pl.BlockSpec((_BN, H), lambda i: (i, 0)),
            pl.BlockSpec((4, H, PH), lambda i: (0, 0, 0)),
            pl.BlockSpec((4, PH), lambda i: (0, 0)),
        ],
        out_specs=pl.BlockSpec((4, _BN, PH), lambda i: (0, i, 0)),
        out_shape=jax.ShapeDtypeStruct((4, N, PH), _F32),
    )(h, w4, b4)


def _step_body(a2_ref, h_ref, wihTl_ref, wihTh_ref, bih_ref,
               whhT_ref, bhh_ref, w_ref, b_ref, hout_ref, tout_ref):
    al = a2_ref[0]
    ah = a2_ref[1]
    h = h_ref[...]

    def gi(k):
        return (_mm(al, wihTl_ref[k]) + _mm(ah, wihTh_ref[k])
                + bih_ref[k:k + 1])

    def gh(k):
        return _mm(h, whhT_ref[k]) + bhh_ref[k:k + 1]

    r = jax.nn.sigmoid(gi(0) + gh(0))
    z = jax.nn.sigmoid(gi(1) + gh(1))
    ng = jnp.tanh(gi(2) + r * gh(2))
    hnew = (1.0 - z) * ng + z * h
    hout_ref[...] = hnew
    _emit_T(hnew, w_ref, b_ref, tout_ref)


def _step_tc(a2, h, wihTl, wihTh, bih, whhT, bhh, w4, b4):
    full = lambda *shape: pl.BlockSpec(shape, lambda i: (0,) * len(shape))
    return pl.pallas_call(
        _step_body,
        grid=(N // _BN,),
        in_specs=[
            pl.BlockSpec((NC, _BN, PH), lambda i: (0, i, 0)),
            pl.BlockSpec((_BN, H), lambda i: (i, 0)),
            full(3, PH, H), full(3, PH, H), full(3, H),
            full(3, H, H), full(3, H),
            full(4, H, PH), full(4, PH),
        ],
        out_specs=[
            pl.BlockSpec((_BN, H), lambda i: (i, 0)),
            pl.BlockSpec((4, _BN, PH), lambda i: (0, i, 0)),
        ],
        out_shape=[
            jax.ShapeDtypeStruct((N, H), _F32),
            jax.ShapeDtypeStruct((4, N, PH), _F32),
        ],
    )(a2, h, wihTl, wihTh, bih, whhT, bhh, w4, b4)


def _pool_mm(C, tin, tout, k, stride):
    col = lax.broadcasted_iota(jnp.int32, (tout, tin), 1)
    row = lax.broadcasted_iota(jnp.int32, (tout, tin), 0)
    out = None
    for kk in range(k):
        sel = (col == stride * row + kk).astype(_F32)
        v = _mm(sel, C)
        out = v if out is None else jnp.maximum(out, v)
    return out


def _head1_body(hi_ref, ci_ref, c1t_ref, c1b_ref, c2t_ref, c2b_ref,
                cc1t_ref, cc1b_ref, cc2t_ref, cc2b_ref,
                myT_ref, myb_ref, mzT_ref, mzb_ref, avg_ref):
    def branch(X, w1t_ref, b1_ref, w2t_ref, b2_ref, mT_ref, mb_ref):
        C = jnp.maximum(
            _mm(X[0:398], w1t_ref[0]) + _mm(X[1:399], w1t_ref[1])
            + _mm(X[2:400], w1t_ref[2]) + b1_ref[...], 0.0)
        P = _pool_mm(C, 398, 198, 3, 2)
        C2 = jnp.maximum(_mm(P, w2t_ref[...]) + b2_ref[...], 0.0)
        Z = _pool_mm(C2, 198, 99, 2, 2)
        return _mm(Z, mT_ref[...]) + mb_ref[...]

    y = branch(hi_ref[0], c1t_ref, c1b_ref, c2t_ref, c2b_ref, myT_ref, myb_ref)
    z = branch(ci_ref[0], cc1t_ref, cc1b_ref, cc2t_ref, cc2b_ref, mzT_ref, mzb_ref)
    avg_ref[0] = jnp.mean(y * z, axis=0, keepdims=True)


def _head1(hi, ci, c1t, c1b, c2t, c2b, cc1t, cc1b, cc2t, cc2b, myT, myb, mzT, mzb):
    G = hi.shape[0]
    full = lambda *shape: pl.BlockSpec(shape, lambda i: (0,) * len(shape))
    return pl.pallas_call(
        _head1_body,
        grid=(G,),
        in_specs=[
            pl.BlockSpec((1, 400, H), lambda i: (i, 0, 0)),
            pl.BlockSpec((1, 400, CONCAT), lambda i: (i, 0, 0)),
            full(3, H, H), full(1, H), full(H, H), full(1, H),
            full(3, CONCAT, CONCAT), full(1, CONCAT),
            full(CONCAT, CONCAT), full(1, CONCAT),
            full(H, 256), full(1, 256), full(CONCAT, 256), full(1, 256),
        ],
        out_specs=pl.BlockSpec((1, 1, 256), lambda i: (i, 0, 0)),
        out_shape=jax.ShapeDtypeStruct((G, 1, 256), _F32),
    )(hi, ci, c1t, c1b, c2t, c2b, cc1t, cc1b, cc2t, cc2b, myT, myb, mzT, mzb)


def _head2_body(avg_ref, l1T_ref, l1b_ref, f1T_ref, f1b_ref, f2T_ref, f2b_ref,
                clsT_ref, clsb_ref, logits_ref, emb_ref):
    o1 = jnp.maximum(_mm(avg_ref[...], l1T_ref[...]) + l1b_ref[...], 0.0)
    ft = jnp.maximum(_mm(o1, f1T_ref[...]) + f1b_ref[...], 0.0)
    em = jnp.maximum(_mm(ft, f2T_ref[...]) + f2b_ref[...], 0.0)
    emb_ref[...] = em
    logits_ref[...] = _mm(em, clsT_ref[...]) + clsb_ref[...]


def _head2(avg, l1T, l1b, f1T, f1b, f2T, f2b, clsT, clsb):
    G = avg.shape[0]
    return pl.pallas_call(
        _head2_body,
        out_shape=[
            jax.ShapeDtypeStruct((G, 2), _F32),
            jax.ShapeDtypeStruct((G, 128), _F32),
        ],
    )(avg, l1T, l1b, f1T, f1b, f2T, f2b, clsT, clsb)


def kernel(x, edge_index, edge_types, ggnn_W, ggnn_b, gru_w_ih, gru_b_ih,
           gru_w_hh, gru_b_hh, conv1_w, conv1_b, conv2_w, conv2_b,
           cconv1_w, cconv1_b, cconv2_w, cconv2_b, mlpy_w, mlpy_b,
           mlpz_w, mlpz_b, l1_w, l1_b, f1_w, f1_b, f2_w, f2_b, cls_w, cls_b):
    g16 = (edge_types * N + edge_index[0]).reshape(NS, NI, K)
    gidx = jnp.concatenate([g16[None], g16[None] + 2 * N], 0).reshape(NW, NI, K)
    d16 = edge_index[1].reshape(NS, NI, K)
    dst = jnp.concatenate([d16[None], d16[None]], 0).reshape(NW, NI, K)

    h = jnp.concatenate([x, jnp.zeros((N, H - IN_F), x.dtype)], axis=1)

    w0t = ggnn_W[0].T
    w1t = ggnn_W[1].T
    w4 = jnp.pad(jnp.stack([w0t[:, :HH], w1t[:, :HH], w0t[:, HH:], w1t[:, HH:]]),
                 ((0, 0), (0, 0), (0, PH - HH)))
    b4 = jnp.pad(jnp.stack([ggnn_b[0][:HH], ggnn_b[1][:HH],
                            ggnn_b[0][HH:], ggnn_b[1][HH:]]),
                 ((0, 0), (0, PH - HH)))
    wihT = jnp.transpose(gru_w_ih.reshape(3, H, H), (0, 2, 1))
    wihTl = jnp.pad(wihT[:, :HH, :], ((0, 0), (0, PH - HH), (0, 0)))
    wihTh = jnp.pad(wihT[:, HH:, :], ((0, 0), (0, PH - HH), (0, 0)))
    bih = gru_b_ih.reshape(3, H)
    whhT = jnp.transpose(gru_w_hh.reshape(3, H, H), (0, 2, 1))
    bhh = gru_b_hh.reshape(3, H)

    T = _tinit(h, w4, b4)
    for _ in range(STEPS):
        a2 = _get_sc_scatter()(T.reshape(4 * N, PH), gidx, dst)
        h, T = _step_tc(a2, h, wihTl, wihTh, bih, whhT, bhh, w4, b4)

    hi = h.reshape(-1, 400, H)
    ci = jnp.concatenate([x, h], axis=1).reshape(-1, 400, CONCAT)

    c1t = jnp.transpose(conv1_w, (2, 1, 0))
    c2t = conv2_w[:, :, 0].T
    cc1t = jnp.transpose(cconv1_w, (2, 1, 0))
    cc2t = cconv2_w[:, :, 0].T
    avg = _head1(hi, ci,
                 c1t, conv1_b[None, :], c2t, conv2_b[None, :],
                 cc1t, cconv1_b[None, :], cc2t, cconv2_b[None, :],
                 mlpy_w.T, mlpy_b[None, :], mlpz_w.T, mlpz_b[None, :]
                 ).reshape(-1, 256)
    logits, emb = _head2(avg, l1_w.T, l1_b[None, :], f1_w.T, f1_b[None, :],
                         f2_w.T, f2_b[None, :], cls_w.T, cls_b[None, :])
    return (logits, emb)


# speedup vs baseline: 7.7040x; 2.1861x over previous
import functools

import jax
import jax.numpy as jnp
from jax import lax
from jax.experimental import pallas as pl
from jax.experimental.pallas import tpu as pltpu
from jax.experimental.pallas import tpu_sc as plsc

N = 10000
E = 320000
IN_F = 128
H = 200
HH = H // 2
PH = 112
STEPS = 8
CONCAT = IN_F + H

NC = 2
NS = 16
NW = NC * NS
EPT = E // NS
K = 128
NI = -(-EPT // K)
EPT_PAD = NI * K
ROW_BLK = 632
ACC_ROWS = NS * ROW_BLK
LAST_ROWS = N - (NS - 1) * ROW_BLK
SINK = N + 64
GIDX_BITS = 17

_F32 = jnp.float32


def _mm(a, b):
    return lax.dot_general(a, b, (((1,), (0,)), ((), ())),
                           preferred_element_type=_F32)


def _sc_scatter_body(t_hbm, meta_hbm, out_hbm,
                     acc, metab, mg0, mg1, md0, md1, st0, st1, sem0, sem1):
    c = lax.axis_index("c")
    s = lax.axis_index("s")
    wid = c * NS + s
    row0 = pl.multiple_of(s * ROW_BLK, 8)
    mg = (mg0, mg1)
    md = (md0, md1)
    st = (st0, st1)
    sem = (sem0, sem1)

    pltpu.sync_copy(meta_hbm.at[wid], metab)

    def zr(r, carry):
        for j in range(PH // 16):
            st0[r, pl.ds(j * 16, 16)] = jnp.zeros((16,), _F32)
        return carry
    lax.fori_loop(0, K, zr, 0)
    for t in range(ROW_BLK // K):
        pltpu.sync_copy(st0, acc.at[pl.ds(row0 + t * K, K)])
    rem = ROW_BLK % K
    if rem:
        pltpu.sync_copy(st0.at[pl.ds(0, rem)],
                        acc.at[pl.ds(row0 + (ROW_BLK // K) * K, rem)])
    plsc.subcore_barrier()

    def start(i, b):
        for j in range(K // 16):
            v = metab[pl.ds(i * K + j * 16, 16)]
            sl = pl.ds(j * 16, 16)
            mg[b][sl] = v & ((1 << GIDX_BITS) - 1)
            md[b][sl] = lax.shift_right_logical(v, GIDX_BITS)
        pltpu.async_copy(t_hbm.at[mg[b]], st[b], sem[b])

    for b in range(2):
        start(b, b)

    def outer(o, carry):
        for b in range(2):
            i = o * 2 + b
            @pl.when(i < NI)
            def _():
                pltpu.make_async_copy(t_hbm.at[mg[b]], st[b], sem[b]).wait()
                pltpu.sync_copy(st[b], acc.at[md[b]], add=True)
                @pl.when(i + 2 < NI)
                def _():
                    start(i + 2, b)
        return carry
    lax.fori_loop(0, (NI + 1) // 2, outer, 0)
    plsc.subcore_barrier()

    @pl.when(s < NS - 1)
    def _():
        pltpu.sync_copy(acc.at[pl.ds(row0, ROW_BLK)],
                        out_hbm.at[c, pl.ds(row0, ROW_BLK)])
    @pl.when(s == NS - 1)
    def _():
        pltpu.sync_copy(acc.at[pl.ds((NS - 1) * ROW_BLK, LAST_ROWS)],
                        out_hbm.at[c, pl.ds((NS - 1) * ROW_BLK, LAST_ROWS)])


@functools.cache
def _get_sc_scatter():
    return pl.kernel(
        _sc_scatter_body,
        out_type=jax.ShapeDtypeStruct((NC, N, PH), _F32),
        mesh=plsc.VectorSubcoreMesh(core_axis_name="c", subcore_axis_name="s",
                                    num_cores=NC, num_subcores=NS),
        scratch_types=[
            pltpu.VMEM_SHARED((ACC_ROWS, PH), _F32),
            pltpu.VMEM((EPT_PAD,), jnp.int32),
            pltpu.VMEM((K,), jnp.int32),
            pltpu.VMEM((K,), jnp.int32),
            pltpu.VMEM((K,), jnp.int32),
            pltpu.VMEM((K,), jnp.int32),
            pltpu.VMEM((K, PH), _F32),
            pltpu.VMEM((K, PH), _F32),
            pltpu.SemaphoreType.DMA,
            pltpu.SemaphoreType.DMA,
        ],
        compiler_params=pltpu.CompilerParams(use_tc_tiling_on_sc=False),
    )


_BN = 2000


def _emit_T(hnew, w_ref, b_ref, tout_ref):
    for q in range(4):
        tout_ref[q] = _mm(hnew, w_ref[q]) + b_ref[q:q + 1]


def _tinit_body(h_ref, w_ref, b_ref, tout_ref):
    _emit_T(h_ref[...], w_ref, b_ref, tout_ref)


def _tinit(h, w4, b4):
    return pl.pallas_call(
        _tinit_body,
        grid=(N // _BN,),
        in_specs=[
            pl.BlockSpec((_BN, H), lambda i: (i, 0)),
            pl.BlockSpec((4, H, PH), lambda i: (0, 0, 0)),
            pl.BlockSpec((4, PH), lambda i: (0, 0)),
        ],
        out_specs=pl.BlockSpec((4, _BN, PH), lambda i: (0, i, 0)),
        out_shape=jax.ShapeDtypeStruct((4, N, PH), _F32),
    )(h, w4, b4)


def _step_body(a2_ref, h_ref, wihTl_ref, wihTh_ref, bih_ref,
               whhT_ref, bhh_ref, w_ref, b_ref, hout_ref, tout_ref):
    al = a2_ref[0]
    ah = a2_ref[1]
    h = h_ref[...]

    def gi(k):
        return (_mm(al, wihTl_ref[k]) + _mm(ah, wihTh_ref[k])
                + bih_ref[k:k + 1])

    def gh(k):
        return _mm(h, whhT_ref[k]) + bhh_ref[k:k + 1]

    r = jax.nn.sigmoid(gi(0) + gh(0))
    z = jax.nn.sigmoid(gi(1) + gh(1))
    ng = jnp.tanh(gi(2) + r * gh(2))
    hnew = (1.0 - z) * ng + z * h
    hout_ref[...] = hnew
    _emit_T(hnew, w_ref, b_ref, tout_ref)


def _step_tc(a2, h, wihTl, wihTh, bih, whhT, bhh, w4, b4):
    full = lambda *shape: pl.BlockSpec(shape, lambda i: (0,) * len(shape))
    return pl.pallas_call(
        _step_body,
        grid=(N // _BN,),
        in_specs=[
            pl.BlockSpec((NC, _BN, PH), lambda i: (0, i, 0)),
            pl.BlockSpec((_BN, H), lambda i: (i, 0)),
            full(3, PH, H), full(3, PH, H), full(3, H),
            full(3, H, H), full(3, H),
            full(4, H, PH), full(4, PH),
        ],
        out_specs=[
            pl.BlockSpec((_BN, H), lambda i: (i, 0)),
            pl.BlockSpec((4, _BN, PH), lambda i: (0, i, 0)),
        ],
        out_shape=[
            jax.ShapeDtypeStruct((N, H), _F32),
            jax.ShapeDtypeStruct((4, N, PH), _F32),
        ],
    )(a2, h, wihTl, wihTh, bih, whhT, bhh, w4, b4)


def _pool_mm(C, tin, tout, k, stride):
    col = lax.broadcasted_iota(jnp.int32, (tout, tin), 1)
    row = lax.broadcasted_iota(jnp.int32, (tout, tin), 0)
    out = None
    for kk in range(k):
        sel = (col == stride * row + kk).astype(_F32)
        v = _mm(sel, C)
        out = v if out is None else jnp.maximum(out, v)
    return out


def _head1_body(hi_ref, ci_ref, c1t_ref, c1b_ref, c2t_ref, c2b_ref,
                cc1t_ref, cc1b_ref, cc2t_ref, cc2b_ref,
                myT_ref, myb_ref, mzT_ref, mzb_ref, avg_ref):
    def branch(X, w1t_ref, b1_ref, w2t_ref, b2_ref, mT_ref, mb_ref):
        C = jnp.maximum(
            _mm(X[0:398], w1t_ref[0]) + _mm(X[1:399], w1t_ref[1])
            + _mm(X[2:400], w1t_ref[2]) + b1_ref[...], 0.0)
        P = _pool_mm(C, 398, 198, 3, 2)
        C2 = jnp.maximum(_mm(P, w2t_ref[...]) + b2_ref[...], 0.0)
        Z = _pool_mm(C2, 198, 99, 2, 2)
        return _mm(Z, mT_ref[...]) + mb_ref[...]

    y = branch(hi_ref[0], c1t_ref, c1b_ref, c2t_ref, c2b_ref, myT_ref, myb_ref)
    z = branch(ci_ref[0], cc1t_ref, cc1b_ref, cc2t_ref, cc2b_ref, mzT_ref, mzb_ref)
    avg_ref[0] = jnp.mean(y * z, axis=0, keepdims=True)


def _head1(hi, ci, c1t, c1b, c2t, c2b, cc1t, cc1b, cc2t, cc2b, myT, myb, mzT, mzb):
    G = hi.shape[0]
    full = lambda *shape: pl.BlockSpec(shape, lambda i: (0,) * len(shape))
    return pl.pallas_call(
        _head1_body,
        grid=(G,),
        in_specs=[
            pl.BlockSpec((1, 400, H), lambda i: (i, 0, 0)),
            pl.BlockSpec((1, 400, CONCAT), lambda i: (i, 0, 0)),
            full(3, H, H), full(1, H), full(H, H), full(1, H),
            full(3, CONCAT, CONCAT), full(1, CONCAT),
            full(CONCAT, CONCAT), full(1, CONCAT),
            full(H, 256), full(1, 256), full(CONCAT, 256), full(1, 256),
        ],
        out_specs=pl.BlockSpec((1, 1, 256), lambda i: (i, 0, 0)),
        out_shape=jax.ShapeDtypeStruct((G, 1, 256), _F32),
    )(hi, ci, c1t, c1b, c2t, c2b, cc1t, cc1b, cc2t, cc2b, myT, myb, mzT, mzb)


def _head2_body(avg_ref, l1T_ref, l1b_ref, f1T_ref, f1b_ref, f2T_ref, f2b_ref,
                clsT_ref, clsb_ref, logits_ref, emb_ref):
    o1 = jnp.maximum(_mm(avg_ref[...], l1T_ref[...]) + l1b_ref[...], 0.0)
    ft = jnp.maximum(_mm(o1, f1T_ref[...]) + f1b_ref[...], 0.0)
    em = jnp.maximum(_mm(ft, f2T_ref[...]) + f2b_ref[...], 0.0)
    emb_ref[...] = em
    logits_ref[...] = _mm(em, clsT_ref[...]) + clsb_ref[...]


def _head2(avg, l1T, l1b, f1T, f1b, f2T, f2b, clsT, clsb):
    G = avg.shape[0]
    return pl.pallas_call(
        _head2_body,
        out_shape=[
            jax.ShapeDtypeStruct((G, 2), _F32),
            jax.ShapeDtypeStruct((G, 128), _F32),
        ],
    )(avg, l1T, l1b, f1T, f1b, f2T, f2b, clsT, clsb)


def kernel(x, edge_index, edge_types, ggnn_W, ggnn_b, gru_w_ih, gru_b_ih,
           gru_w_hh, gru_b_hh, conv1_w, conv1_b, conv2_w, conv2_b,
           cconv1_w, cconv1_b, cconv2_w, cconv2_b, mlpy_w, mlpy_b,
           mlpz_w, mlpz_b, l1_w, l1_b, f1_w, f1_b, f2_w, f2_b, cls_w, cls_b):
    g16 = jnp.pad((edge_types * N + edge_index[0]).reshape(NS, EPT),
                  ((0, 0), (0, EPT_PAD - EPT)))
    d16 = jnp.pad(edge_index[1].reshape(NS, EPT),
                  ((0, 0), (0, EPT_PAD - EPT)), constant_values=SINK)
    m16 = d16 * (1 << GIDX_BITS) + g16
    meta = jnp.concatenate([m16[None], m16[None] + 2 * N], 0).reshape(NW, EPT_PAD)

    h = jnp.concatenate([x, jnp.zeros((N, H - IN_F), x.dtype)], axis=1)

    w0t = ggnn_W[0].T
    w1t = ggnn_W[1].T
    w4 = jnp.pad(jnp.stack([w0t[:, :HH], w1t[:, :HH], w0t[:, HH:], w1t[:, HH:]]),
                 ((0, 0), (0, 0), (0, PH - HH)))
    b4 = jnp.pad(jnp.stack([ggnn_b[0][:HH], ggnn_b[1][:HH],
                            ggnn_b[0][HH:], ggnn_b[1][HH:]]),
                 ((0, 0), (0, PH - HH)))
    wihT = jnp.transpose(gru_w_ih.reshape(3, H, H), (0, 2, 1))
    wihTl = jnp.pad(wihT[:, :HH, :], ((0, 0), (0, PH - HH), (0, 0)))
    wihTh = jnp.pad(wihT[:, HH:, :], ((0, 0), (0, PH - HH), (0, 0)))
    bih = gru_b_ih.reshape(3, H)
    whhT = jnp.transpose(gru_w_hh.reshape(3, H, H), (0, 2, 1))
    bhh = gru_b_hh.reshape(3, H)

    T = _tinit(h, w4, b4)
    for _ in range(STEPS):
        a2 = _get_sc_scatter()(T.reshape(4 * N, PH), meta)
        h, T = _step_tc(a2, h, wihTl, wihTh, bih, whhT, bhh, w4, b4)

    hi = h.reshape(-1, 400, H)
    ci = jnp.concatenate([x, h], axis=1).reshape(-1, 400, CONCAT)

    c1t = jnp.transpose(conv1_w, (2, 1, 0))
    c2t = conv2_w[:, :, 0].T
    cc1t = jnp.transpose(cconv1_w, (2, 1, 0))
    cc2t = cconv2_w[:, :, 0].T
    avg = _head1(hi, ci,
                 c1t, conv1_b[None, :], c2t, conv2_b[None, :],
                 cc1t, cconv1_b[None, :], cc2t, cconv2_b[None, :],
                 mlpy_w.T, mlpy_b[None, :], mlpz_w.T, mlpz_b[None, :]
                 ).reshape(-1, 256)
    logits, emb = _head2(avg, l1_w.T, l1_b[None, :], f1_w.T, f1_b[None, :],
                         f2_w.T, f2_b[None, :], cls_w.T, cls_b[None, :])
    return (logits, emb)
